# Initial kernel scaffold; baseline (speedup 1.0000x reference)
import jax
import jax.numpy as jnp
from jax.experimental import pallas as pl


def kernel(x, token_table):
    raise NotImplementedError("write your pallas kernel here")


# speedup vs baseline: 3.1795x; 3.1795x over previous
import functools

import jax
import jax.numpy as jnp
import numpy as np
from jax import lax
from jax.experimental import pallas as pl
from jax.experimental.pallas import tpu as pltpu
from jax.experimental.pallas import tpu_sc as plsc

VOCAB = 100000
D_MODEL = 1024
MAX_SEQ = 2048
PAD_IDX = 1
BATCH = 4
SEQ = 2048

NC = 2
NS = 16
NW = NC * NS
N_ROWS = BATCH * SEQ
ROWS_PER_W = N_ROWS // NW
CHUNK = 32
N_CHUNKS = ROWS_PER_W // CHUNK
LANES = 16
SEQ_PER_W = SEQ // ROWS_PER_W if SEQ >= ROWS_PER_W else 0
W_PER_BATCH = SEQ // ROWS_PER_W


def _make_pos_enc() -> np.ndarray:
    pos = np.arange(MAX_SEQ, dtype=np.float32)[:, None]
    i = np.arange(0, D_MODEL, 2, dtype=np.float32)
    div = np.power(10000.0, i / float(D_MODEL))
    pe = np.zeros((MAX_SEQ, D_MODEL), dtype=np.float32)
    pe[:, 0::2] = np.sin(pos / div)
    pe[:, 1::2] = np.cos(pos / div)
    return pe


_PE = _make_pos_enc()[:SEQ]


def _embed_body(x_hbm, tbl_hbm, pe_hbm, out_hbm, idx_v, scale_v, tok_v, pe_v, sem):
    cid = lax.axis_index("c")
    sid = lax.axis_index("s")
    wid = sid * NC + cid
    base = wid * ROWS_PER_W
    seq0 = (wid % W_PER_BATCH) * ROWS_PER_W

    pltpu.sync_copy(x_hbm.at[wid], idx_v)

    def chunk_step(c, carry):
        pltpu.async_copy(tbl_hbm.at[idx_v.at[c]], tok_v, sem).wait()
        pltpu.sync_copy(pe_hbm.at[pl.ds(seq0 + c * CHUNK, CHUNK)], pe_v)

        for g in range(CHUNK // LANES):
            iv = idx_v[c, pl.ds(g * LANES, LANES)]
            scale_v[pl.ds(g * LANES, LANES)] = jnp.where(
                iv == PAD_IDX, 0.0, 1.0).astype(jnp.float32)

        def row_step(r, carry2):
            rid = jnp.broadcast_to(r, (LANES,)).astype(jnp.int32)
            scale = plsc.load_gather(scale_v, [rid])
            for j in range(D_MODEL // LANES):
                sl = pl.ds(j * LANES, LANES)
                tok_v[r, sl] = tok_v[r, sl] * scale + pe_v[r, sl]
            return carry2

        lax.fori_loop(0, CHUNK, row_step, 0)
        pltpu.sync_copy(tok_v, out_hbm.at[pl.ds(base + c * CHUNK, CHUNK)])
        return carry

    lax.fori_loop(0, N_CHUNKS, chunk_step, 0)


@jax.jit
def kernel(x, token_table):
    pe = jnp.asarray(_PE)
    x_r = x.reshape(NW, N_CHUNKS, CHUNK)
    mesh = plsc.VectorSubcoreMesh(core_axis_name="c", subcore_axis_name="s")
    out = pl.kernel(
        _embed_body,
        mesh=mesh,
        compiler_params=pltpu.CompilerParams(needs_layout_passes=False),
        out_type=jax.ShapeDtypeStruct((N_ROWS, D_MODEL), jnp.float32),
        scratch_types=[
            pltpu.VMEM((N_CHUNKS, CHUNK), jnp.int32),
            pltpu.VMEM((CHUNK,), jnp.float32),
            pltpu.VMEM((CHUNK, D_MODEL), jnp.float32),
            pltpu.VMEM((CHUNK, D_MODEL), jnp.float32),
            pltpu.SemaphoreType.DMA,
        ],
    )(x_r, token_table, pe)
    return out.reshape(BATCH, SEQ, D_MODEL)


# speedup vs baseline: 3.5842x; 1.1273x over previous
import jax
import jax.numpy as jnp
import numpy as np
from jax import lax
from jax.experimental import pallas as pl
from jax.experimental.pallas import tpu as pltpu
from jax.experimental.pallas import tpu_sc as plsc

VOCAB = 100000
D_MODEL = 1024
MAX_SEQ = 2048
PAD_IDX = 1
BATCH = 4
SEQ = 2048

NC = 2
NS = 16
NW = NC * NS
SEQ_PER_W = SEQ // NW
SUB = 16
N_SUB = SEQ_PER_W // SUB
N_STEPS = BATCH * N_SUB
LANES = 16


def _make_pos_enc() -> np.ndarray:
    pos = np.arange(MAX_SEQ, dtype=np.float32)[:, None]
    i = np.arange(0, D_MODEL, 2, dtype=np.float32)
    div = np.power(10000.0, i / float(D_MODEL))
    pe = np.zeros((MAX_SEQ, D_MODEL), dtype=np.float32)
    pe[:, 0::2] = np.sin(pos / div)
    pe[:, 1::2] = np.cos(pos / div)
    return pe


_PE = _make_pos_enc()[:SEQ]


def _embed_body(x_hbm, tbl_hbm, pe_hbm, out_hbm,
                idx_v, scale_v, pe_v, tok_v,
                gsem0, gsem1, osem0, osem1, psem):
    cid = lax.axis_index("c")
    sid = lax.axis_index("s")
    wid = sid * NC + cid
    s0 = wid * SEQ_PER_W

    pe_cp = pltpu.async_copy(pe_hbm.at[pl.ds(s0, SEQ_PER_W)], pe_v, psem)

    for b in range(BATCH):
        pltpu.sync_copy(x_hbm.at[b, pl.ds(s0, SEQ_PER_W)], idx_v.at[b])

    for b in range(BATCH):
        for g in range(SEQ_PER_W // LANES):
            iv = idx_v[b, pl.ds(g * LANES, LANES)]
            scale_v[pl.ds(b * SEQ_PER_W + g * LANES, LANES)] = jnp.where(
                iv == PAD_IDX, 0.0, 1.0).astype(jnp.float32)

    gsems = (gsem0, gsem1)
    osems = (osem0, osem1)

    def issue_gather(k):
        b, h = divmod(k, N_SUB)
        return pltpu.async_copy(
            tbl_hbm.at[idx_v.at[b, pl.ds(h * SUB, SUB)]],
            tok_v.at[k % 2], gsems[k % 2])

    def issue_out(k):
        b, h = divmod(k, N_SUB)
        return pltpu.async_copy(
            tok_v.at[k % 2],
            out_hbm.at[pl.ds(b * SEQ + s0 + h * SUB, SUB)], osems[k % 2])

    g_cp = {0: issue_gather(0)}
    o_cp = {}
    for k in range(N_STEPS):
        b, h = divmod(k, N_SUB)
        kb = k % 2
        g_cp[k].wait()
        if k + 1 < N_STEPS:
            if k >= 1:
                o_cp[k - 1].wait()
            g_cp[k + 1] = issue_gather(k + 1)
        if k == 0:
            pe_cp.wait()

        def row_step(r, carry, b=b, h=h, kb=kb):
            rid = jnp.broadcast_to(b * SEQ_PER_W + h * SUB + r,
                                   (LANES,)).astype(jnp.int32)
            scale = plsc.load_gather(scale_v, [rid])
            for j in range(D_MODEL // LANES):
                sl = pl.ds(j * LANES, LANES)
                tok_v[kb, r, sl] = (tok_v[kb, r, sl] * scale
                                    + pe_v[h * SUB + r, sl])
            return carry

        lax.fori_loop(0, SUB, row_step, 0)
        o_cp[k] = issue_out(k)

    o_cp[N_STEPS - 2].wait()
    o_cp[N_STEPS - 1].wait()


@jax.jit
def kernel(x, token_table):
    pe = jnp.asarray(_PE)
    mesh = plsc.VectorSubcoreMesh(core_axis_name="c", subcore_axis_name="s")
    out = pl.kernel(
        _embed_body,
        mesh=mesh,
        compiler_params=pltpu.CompilerParams(needs_layout_passes=False),
        out_type=jax.ShapeDtypeStruct((BATCH * SEQ, D_MODEL), jnp.float32),
        scratch_types=[
            pltpu.VMEM((BATCH, SEQ_PER_W), jnp.int32),
            pltpu.VMEM((BATCH * SEQ_PER_W,), jnp.float32),
            pltpu.VMEM((SEQ_PER_W, D_MODEL), jnp.float32),
            pltpu.VMEM((2, SUB, D_MODEL), jnp.float32),
            pltpu.SemaphoreType.DMA,
            pltpu.SemaphoreType.DMA,
            pltpu.SemaphoreType.DMA,
            pltpu.SemaphoreType.DMA,
            pltpu.SemaphoreType.DMA,
        ],
    )(x, token_table, pe)
    return out.reshape(BATCH, SEQ, D_MODEL)


# speedup vs baseline: 4.2671x; 1.1905x over previous
import jax
import jax.numpy as jnp
import numpy as np
from jax import lax
from jax.experimental import pallas as pl
from jax.experimental.pallas import tpu as pltpu
from jax.experimental.pallas import tpu_sc as plsc

VOCAB = 100000
D_MODEL = 1024
MAX_SEQ = 2048
PAD_IDX = 1
BATCH = 4
SEQ = 2048

NC = 2
NS = 16
NW = NC * NS
SEQ_PER_W = SEQ // NW
SUB = 16
N_SUB = SEQ_PER_W // SUB
N_STEPS = BATCH * N_SUB
LANES = 16


def _make_pos_enc() -> np.ndarray:
    pos = np.arange(MAX_SEQ, dtype=np.float32)[:, None]
    i = np.arange(0, D_MODEL, 2, dtype=np.float32)
    div = np.power(10000.0, i / float(D_MODEL))
    pe = np.zeros((MAX_SEQ, D_MODEL), dtype=np.float32)
    pe[:, 0::2] = np.sin(pos / div)
    pe[:, 1::2] = np.cos(pos / div)
    return pe


_PE = _make_pos_enc()[:SEQ]


def _embed_body(x_hbm, tbl_hbm, pe_hbm, out_hbm,
                idx_v, scale_v, pe_v, tok_v,
                gsem0, gsem1, osem0, osem1, psem):
    cid = lax.axis_index("c")
    sid = lax.axis_index("s")
    wid = sid * NC + cid
    s0 = wid * SEQ_PER_W

    pe_cp = pltpu.async_copy(pe_hbm.at[pl.ds(s0, SEQ_PER_W)], pe_v, psem)

    for b in range(BATCH):
        pltpu.sync_copy(x_hbm.at[b, pl.ds(s0, SEQ_PER_W)], idx_v.at[b])

    for b in range(BATCH):
        for g in range(SEQ_PER_W // LANES):
            iv = idx_v[b, pl.ds(g * LANES, LANES)]
            scale_v[pl.ds(b * SEQ_PER_W + g * LANES, LANES)] = jnp.where(
                iv == PAD_IDX, 0.0, 1.0).astype(jnp.float32)

    gsems = (gsem0, gsem1)
    osems = (osem0, osem1)

    def issue_gather(k):
        b, h = divmod(k, N_SUB)
        return pltpu.async_copy(
            tbl_hbm.at[idx_v.at[b, pl.ds(h * SUB, SUB)]],
            tok_v.at[k % 2], gsems[k % 2])

    def issue_out(k):
        b, h = divmod(k, N_SUB)
        return pltpu.async_copy(
            tok_v.at[k % 2],
            out_hbm.at[pl.ds(b * SEQ + s0 + h * SUB, SUB)], osems[k % 2])

    g_cp = {0: issue_gather(0)}
    o_cp = {}
    for k in range(N_STEPS):
        b, h = divmod(k, N_SUB)
        kb = k % 2
        g_cp[k].wait()
        if k + 1 < N_STEPS:
            if k >= 1:
                o_cp[k - 1].wait()
            g_cp[k + 1] = issue_gather(k + 1)
        if k == 0:
            pe_cp.wait()

        iv = idx_v[b, pl.ds(h * SUB, SUB)]
        padcnt = jnp.sum(jnp.where(iv == PAD_IDX, 1, 0).astype(jnp.int32))

        @pl.when(padcnt > 0)
        def _scale_pass(b=b, h=h, kb=kb):
            def srow(r, carry):
                rid = jnp.broadcast_to(b * SEQ_PER_W + h * SUB + r,
                                       (LANES,)).astype(jnp.int32)
                scale = plsc.load_gather(scale_v, [rid])
                for j in range(D_MODEL // LANES):
                    sl = pl.ds(j * LANES, LANES)
                    tok_v[kb, r, sl] = tok_v[kb, r, sl] * scale
                return carry
            lax.fori_loop(0, SUB, srow, 0)

        def arow(r, carry, h=h, kb=kb):
            for j in range(D_MODEL // LANES):
                sl = pl.ds(j * LANES, LANES)
                plsc.addupdate(tok_v.at[kb, r, sl], pe_v[h * SUB + r, sl])
            return carry

        lax.fori_loop(0, SUB, arow, 0)
        o_cp[k] = issue_out(k)

    o_cp[N_STEPS - 2].wait()
    o_cp[N_STEPS - 1].wait()


@jax.jit
def kernel(x, token_table):
    pe = jnp.asarray(_PE)
    mesh = plsc.VectorSubcoreMesh(core_axis_name="c", subcore_axis_name="s")
    out = pl.kernel(
        _embed_body,
        mesh=mesh,
        compiler_params=pltpu.CompilerParams(needs_layout_passes=False),
        out_type=jax.ShapeDtypeStruct((BATCH * SEQ, D_MODEL), jnp.float32),
        scratch_types=[
            pltpu.VMEM((BATCH, SEQ_PER_W), jnp.int32),
            pltpu.VMEM((BATCH * SEQ_PER_W,), jnp.float32),
            pltpu.VMEM((SEQ_PER_W, D_MODEL), jnp.float32),
            pltpu.VMEM((2, SUB, D_MODEL), jnp.float32),
            pltpu.SemaphoreType.DMA,
            pltpu.SemaphoreType.DMA,
            pltpu.SemaphoreType.DMA,
            pltpu.SemaphoreType.DMA,
            pltpu.SemaphoreType.DMA,
        ],
    )(x, token_table, pe)
    return out.reshape(BATCH, SEQ, D_MODEL)


# speedup vs baseline: 5.1874x; 1.2157x over previous
import jax
import jax.numpy as jnp
import numpy as np
from jax import lax
from jax.experimental import pallas as pl
from jax.experimental.pallas import tpu as pltpu
from jax.experimental.pallas import tpu_sc as plsc

VOCAB = 100000
D_MODEL = 1024
MAX_SEQ = 2048
PAD_IDX = 1
BATCH = 4
SEQ = 2048

NC = 2
NS = 16
NW = NC * NS
SEQ_PER_W = SEQ // NW
SUB = 8
N_STEPS = SEQ_PER_W // SUB
LANES = 16
NCH = D_MODEL // LANES


def _make_pos_enc() -> np.ndarray:
    pos = np.arange(MAX_SEQ, dtype=np.float32)[:, None]
    i = np.arange(0, D_MODEL, 2, dtype=np.float32)
    div = np.power(10000.0, i / float(D_MODEL))
    pe = np.zeros((MAX_SEQ, D_MODEL), dtype=np.float32)
    pe[:, 0::2] = np.sin(pos / div)
    pe[:, 1::2] = np.cos(pos / div)
    return pe


_PE = _make_pos_enc()[:SEQ]


def _embed_body(x_hbm, tbl_hbm, pe_hbm, out_hbm,
                idx_v, scale_v, tok_v, pe_v,
                gsem0, gsem1, osem0, osem1, psem0, psem1):
    cid = lax.axis_index("c")
    sid = lax.axis_index("s")
    wid = sid * NC + cid
    s0 = wid * SEQ_PER_W

    for b in range(BATCH):
        pltpu.sync_copy(x_hbm.at[b, pl.ds(s0, SEQ_PER_W)], idx_v.at[b])

    padcnt = jnp.int32(0)
    for b in range(BATCH):
        for g in range(SEQ_PER_W // LANES):
            iv = idx_v[b, pl.ds(g * LANES, LANES)]
            hit = jnp.where(iv == PAD_IDX, 1, 0).astype(jnp.int32)
            scale_v[pl.ds(b * SEQ_PER_W + g * LANES, LANES)] = (
                1.0 - hit.astype(jnp.float32))
            padcnt = padcnt + jnp.sum(hit)

    gsems = (gsem0, gsem1)
    osems = (osem0, osem1)
    psems = (psem0, psem1)

    def issue_gathers(k):
        return [pltpu.async_copy(
            tbl_hbm.at[idx_v.at[b, pl.ds(k * SUB, SUB)]],
            tok_v.at[k % 2, b], gsems[k % 2]) for b in range(BATCH)]

    def issue_pe(k):
        return pltpu.async_copy(
            pe_hbm.at[pl.ds(s0 + k * SUB, SUB)], pe_v.at[k % 2], psems[k % 2])

    def issue_outs(k):
        return [pltpu.async_copy(
            tok_v.at[k % 2, b],
            out_hbm.at[pl.ds(b * SEQ + s0 + k * SUB, SUB)], osems[k % 2])
            for b in range(BATCH)]

    g_cp = {0: issue_gathers(0)}
    p_cp = {0: issue_pe(0)}
    o_cp = {}
    for k in range(N_STEPS):
        kb = k % 2
        for d in g_cp[k]:
            d.wait()
        p_cp[k].wait()
        if k + 1 < N_STEPS:
            if k >= 1:
                for d in o_cp[k - 1]:
                    d.wait()
            g_cp[k + 1] = issue_gathers(k + 1)
            p_cp[k + 1] = issue_pe(k + 1)

        @pl.when(padcnt > 0)
        def _scale_pass(k=k, kb=kb):
            def srow(r, carry):
                b = r // SUB
                sp = r % SUB
                rid = jnp.broadcast_to(b * SEQ_PER_W + k * SUB + sp,
                                       (LANES,)).astype(jnp.int32)
                scale = plsc.load_gather(scale_v, [rid])
                for j in range(NCH):
                    sl = pl.ds(j * LANES, LANES)
                    tok_v[kb, b, sp, sl] = tok_v[kb, b, sp, sl] * scale
                return carry
            lax.fori_loop(0, BATCH * SUB, srow, 0)

        def arow(sp, carry, kb=kb):
            for j in range(NCH):
                sl = pl.ds(j * LANES, LANES)
                pvec = pe_v[kb, sp, sl]
                for b in range(BATCH):
                    plsc.addupdate(tok_v.at[kb, b, sp, sl], pvec)
            return carry

        lax.fori_loop(0, SUB, arow, 0)
        o_cp[k] = issue_outs(k)

    for d in o_cp[N_STEPS - 2]:
        d.wait()
    for d in o_cp[N_STEPS - 1]:
        d.wait()


@jax.jit
def kernel(x, token_table):
    pe = jnp.asarray(_PE)
    mesh = plsc.VectorSubcoreMesh(core_axis_name="c", subcore_axis_name="s")
    out = pl.kernel(
        _embed_body,
        mesh=mesh,
        compiler_params=pltpu.CompilerParams(needs_layout_passes=False),
        out_type=jax.ShapeDtypeStruct((BATCH * SEQ, D_MODEL), jnp.float32),
        scratch_types=[
            pltpu.VMEM((BATCH, SEQ_PER_W), jnp.int32),
            pltpu.VMEM((BATCH * SEQ_PER_W,), jnp.float32),
            pltpu.VMEM((2, BATCH, SUB, D_MODEL), jnp.float32),
            pltpu.VMEM((2, SUB, D_MODEL), jnp.float32),
            pltpu.SemaphoreType.DMA,
            pltpu.SemaphoreType.DMA,
            pltpu.SemaphoreType.DMA,
            pltpu.SemaphoreType.DMA,
            pltpu.SemaphoreType.DMA,
            pltpu.SemaphoreType.DMA,
        ],
    )(x, token_table, pe)
    return out.reshape(BATCH, SEQ, D_MODEL)
